# baseline (speedup 1.0000x reference)
import numpy as np
import jax
import jax.numpy as jnp
from jax.experimental import pallas as pl

D_MODEL = 1024
RANK = 128
N_INPUT = 8
N_PROCESS = 32
N_OUTPUT = 8
K = 4

TILE = 2048

_E_MAT = np.repeat(np.eye(N_INPUT, dtype=np.float32), RANK, axis=1).astype(jnp.bfloat16)
_G_MAT = np.tile(np.eye(RANK, dtype=np.float32), (N_INPUT, 1)).astype(jnp.bfloat16)
_GREP_MAT = np.tile(np.eye(RANK, dtype=np.float32), (1, N_OUTPUT)).astype(jnp.bfloat16)


def _body(x_ref, wi_ref, idx_ref, wo_ref, win_ref, pn_ref, won_ref,
          e_ref, g_ref, grep_ref, out_ref):
    T = x_ref.shape[0]
    f32 = jnp.float32
    bf16 = jnp.bfloat16

    P = jnp.dot(x_ref[...].astype(bf16), win_ref[...], preferred_element_type=f32)
    wiexp = jnp.dot(wi_ref[...].astype(bf16), e_ref[...], preferred_element_type=f32)
    h = jnp.dot((P * wiexp).astype(bf16), g_ref[...], preferred_element_type=f32)

    pn = pn_ref[...]
    pnhat = pn * jax.lax.rsqrt(jnp.sum(pn * pn, axis=1, keepdims=True) + 1e-8)
    pnhat16 = pnhat.astype(bf16)
    pnhatT16 = pnhat.T.astype(bf16)
    idx = idx_ref[...]
    iota = jax.lax.broadcasted_iota(jnp.int32, (T, N_PROCESS), 1)
    for i in range(K):
        g = jnp.dot(h.astype(bf16), pnhatT16, preferred_element_type=f32)
        oh = idx[:, i : i + 1] == iota
        sel = jnp.where(oh, g * 2.0, 0.0)
        h = h - jnp.dot(sel.astype(bf16), pnhat16, preferred_element_type=f32)

    woexp = jnp.dot(wo_ref[...].astype(bf16), e_ref[...], preferred_element_type=f32)
    hrep = jnp.dot(h.astype(bf16), grep_ref[...], preferred_element_type=f32)
    out_ref[...] = jnp.dot((hrep * woexp).astype(bf16), won_ref[...],
                           preferred_element_type=f32)


def kernel(x, input_weights, process_indices, output_weights,
           input_neurons, process_neurons, output_neurons):
    B, S, D = x.shape
    N = B * S
    xf = x.reshape(N, D)
    wif = input_weights.reshape(N, N_INPUT)
    idxf = process_indices.reshape(N, K).astype(jnp.int32)
    wof = output_weights.reshape(N, N_OUTPUT)
    win16 = jnp.transpose(input_neurons, (1, 0, 2)).reshape(D, N_INPUT * RANK).astype(jnp.bfloat16)
    won16 = output_neurons.reshape(N_OUTPUT * RANK, D).astype(jnp.bfloat16)

    grid = (N // TILE,)
    out = pl.pallas_call(
        _body,
        grid=grid,
        in_specs=[
            pl.BlockSpec((TILE, D), lambda i: (i, 0)),
            pl.BlockSpec((TILE, N_INPUT), lambda i: (i, 0)),
            pl.BlockSpec((TILE, K), lambda i: (i, 0)),
            pl.BlockSpec((TILE, N_OUTPUT), lambda i: (i, 0)),
            pl.BlockSpec((D, N_INPUT * RANK), lambda i: (0, 0)),
            pl.BlockSpec((N_PROCESS, RANK), lambda i: (0, 0)),
            pl.BlockSpec((N_OUTPUT * RANK, D), lambda i: (0, 0)),
            pl.BlockSpec((N_INPUT, N_INPUT * RANK), lambda i: (0, 0)),
            pl.BlockSpec((N_INPUT * RANK, RANK), lambda i: (0, 0)),
            pl.BlockSpec((RANK, N_OUTPUT * RANK), lambda i: (0, 0)),
        ],
        out_specs=pl.BlockSpec((TILE, D), lambda i: (i, 0)),
        out_shape=jax.ShapeDtypeStruct((N, D), jnp.float32),
    )(xf, wif, idxf, wof, win16, process_neurons, won16,
      jnp.asarray(_E_MAT), jnp.asarray(_G_MAT), jnp.asarray(_GREP_MAT))
    return out.reshape(B, S, D)


# speedup vs baseline: 1.3620x; 1.3620x over previous
import numpy as np
import jax
import jax.numpy as jnp
from jax.experimental import pallas as pl

D_MODEL = 1024
RANK = 128
N_INPUT = 8
N_PROCESS = 32
N_OUTPUT = 8
K = 4

TILE = 1024

def _body(x_ref, wi_ref, idx_ref, wo_ref, win_ref, pn_ref, won_ref, out_ref):
    T = x_ref.shape[0]
    f32 = jnp.float32
    bf16 = jnp.bfloat16

    P = jnp.dot(x_ref[...].astype(bf16), win_ref[...], preferred_element_type=f32)
    wi = wi_ref[...]
    h = P[:, :RANK] * wi[:, 0:1]
    for n in range(1, N_INPUT):
        h = h + P[:, n * RANK:(n + 1) * RANK] * wi[:, n : n + 1]

    pn = pn_ref[...]
    pnhat = pn * jax.lax.rsqrt(jnp.sum(pn * pn, axis=1, keepdims=True) + 1e-8)
    pnhat2_16 = (pnhat * 2.0).astype(bf16)
    pnhatT16 = pnhat.T.astype(bf16)
    idx = idx_ref[...]
    iota = jax.lax.broadcasted_iota(jnp.int32, (T, N_PROCESS), 1)
    for i in range(K):
        g = jnp.dot(h.astype(bf16), pnhatT16, preferred_element_type=f32)
        oh = idx[:, i : i + 1] == iota
        sel = jnp.where(oh, g, 0.0)
        h = h - jnp.dot(sel.astype(bf16), pnhat2_16, preferred_element_type=f32)

    h16 = h.astype(bf16)
    wo16 = wo_ref[...].astype(bf16)
    hw = jnp.concatenate(
        [h16 * wo16[:, n : n + 1] for n in range(N_OUTPUT)], axis=1)
    out_ref[...] = jnp.dot(hw, won_ref[...], preferred_element_type=f32)


def kernel(x, input_weights, process_indices, output_weights,
           input_neurons, process_neurons, output_neurons):
    B, S, D = x.shape
    N = B * S
    xf = x.reshape(N, D)
    wif = input_weights.reshape(N, N_INPUT)
    idxf = process_indices.reshape(N, K).astype(jnp.int32)
    wof = output_weights.reshape(N, N_OUTPUT)
    win16 = jnp.transpose(input_neurons, (1, 0, 2)).reshape(D, N_INPUT * RANK).astype(jnp.bfloat16)
    won16 = output_neurons.reshape(N_OUTPUT * RANK, D).astype(jnp.bfloat16)

    grid = (N // TILE,)
    out = pl.pallas_call(
        _body,
        grid=grid,
        in_specs=[
            pl.BlockSpec((TILE, D), lambda i: (i, 0)),
            pl.BlockSpec((TILE, N_INPUT), lambda i: (i, 0)),
            pl.BlockSpec((TILE, K), lambda i: (i, 0)),
            pl.BlockSpec((TILE, N_OUTPUT), lambda i: (i, 0)),
            pl.BlockSpec((D, N_INPUT * RANK), lambda i: (0, 0)),
            pl.BlockSpec((N_PROCESS, RANK), lambda i: (0, 0)),
            pl.BlockSpec((N_OUTPUT * RANK, D), lambda i: (0, 0)),
        ],
        out_specs=pl.BlockSpec((TILE, D), lambda i: (i, 0)),
        out_shape=jax.ShapeDtypeStruct((N, D), jnp.float32),
    )(xf, wif, idxf, wof, win16, process_neurons, won16)
    return out.reshape(B, S, D)
